# baseline (device time: 28669 ns/iter reference)
import jax
import jax.numpy as jnp
from jax import lax
from jax.experimental import pallas as pl
from jax.experimental.pallas import tpu as pltpu

N_DEV = 16
N_PARTS = 4
N_STEPS = 6

ORDERS = [
    [0, 1, 3, 2],
    [1, 2, 3, 0],
    [2, 0, 1, 3],
    [0, 2, 1, 3],
]


def kernel(ids, E):
    v_per, d = E.shape
    t = ids.shape[0]
    rows_per_part = t // N_PARTS
    h0 = rows_per_part // 2
    h1 = rows_per_part // 4

    my = lax.axis_index("i")
    local = ids - my * v_per
    mask = (local >= 0) & (local < v_per)
    safe = jnp.where(mask, local, 0)
    gathered = E[safe]
    maskf = mask.astype(jnp.float32)[:, None]

    def body(p_ref, m_ref, out_ref, s0, s1, s2, s3, send_sems, recv_sems):
        idx = lax.axis_index("i")
        z = idx // 4
        p = idx % 4
        x = jnp.where((p == 1) | (p == 2), 1, 0)
        y = jnp.where(p >= 2, 1, 0)

        def to_idx(xx, yy, zz):
            pp = jnp.where(yy == 1, 3 - xx, xx)
            return zz * 4 + pp

        partners = [
            to_idx(1 - x, y, z),
            to_idx(x, 1 - y, z),
            to_idx(x, y, jnp.bitwise_xor(z, 1)),
            to_idx(x, y, jnp.bitwise_xor(z, 2)),
        ]
        bits = [x, y, jnp.bitwise_and(z, 1), z // 2]
        scratches = [s0, s1, s2, s3]

        barrier_sem = pltpu.get_barrier_semaphore()
        for nbr in partners:
            pl.semaphore_signal(
                barrier_sem, inc=1,
                device_id=(nbr,), device_id_type=pl.DeviceIdType.MESH,
            )
        pl.semaphore_wait(barrier_sem, 4)

        out_ref[:, :] = out_ref[:, :] * m_ref[:, :]

        def sem_idx(s, a):
            return s * N_PARTS + a

        off = [jnp.int32(a * rows_per_part) for a in range(N_PARTS)]

        def start(step, a):
            if step == 0 or step == 1:
                half = h0 if step == 0 else h1
                ax = ORDERS[a][step]
                b = bits[ax]
                my_off = off[a] + b * half
                pr_off = off[a] + (1 - b) * half
                src = out_ref.at[pl.ds(pr_off, half), :]
                dst = scratches[step].at[a]
                off[a] = my_off
                add_off = my_off
            elif step == 2 or step == 3:
                ax = ORDERS[a][step]
                src = out_ref.at[pl.ds(off[a], h1), :]
                dst = scratches[step].at[a]
                add_off = off[a]
            else:
                slot = 5 - step
                half = h1 if slot == 1 else h0
                ax = ORDERS[a][slot]
                src = out_ref.at[pl.ds(off[a], half), :]
                dst = out_ref.at[pl.ds(off[a], half), :]
                off[a] = off[a] - bits[ax] * half
                add_off = None
            rdma = pltpu.make_async_remote_copy(
                src_ref=src,
                dst_ref=dst,
                send_sem=send_sems.at[sem_idx(step, a)],
                recv_sem=recv_sems.at[sem_idx(step, a)],
                device_id=(partners[ax],),
                device_id_type=pl.DeviceIdType.MESH,
            )
            rdma.start()
            return rdma, add_off

        n_rows = [h0, h1, h1, h1, h1, h0]
        inflight = [start(0, a) for a in range(N_PARTS)]
        for step in range(1, N_STEPS):
            for a in range(N_PARTS):
                rdma, add_off = inflight[a]
                rdma.wait()
                if add_off is not None:
                    out_ref[pl.ds(add_off, n_rows[step - 1]), :] += (
                        scratches[step - 1][a, :, :]
                    )
                inflight[a] = start(step, a)
        for a in range(N_PARTS):
            inflight[a][0].wait()

    return pl.pallas_call(
        body,
        out_shape=jax.ShapeDtypeStruct((t, d), jnp.float32),
        in_specs=[
            pl.BlockSpec(memory_space=pltpu.VMEM),
            pl.BlockSpec(memory_space=pltpu.VMEM),
        ],
        out_specs=pl.BlockSpec(memory_space=pltpu.VMEM),
        input_output_aliases={0: 0},
        scratch_shapes=[
            pltpu.VMEM((N_PARTS, h0, d), jnp.float32),
            pltpu.VMEM((N_PARTS, h1, d), jnp.float32),
            pltpu.VMEM((N_PARTS, h1, d), jnp.float32),
            pltpu.VMEM((N_PARTS, h1, d), jnp.float32),
            pltpu.SemaphoreType.DMA((N_STEPS * N_PARTS,)),
            pltpu.SemaphoreType.DMA((N_STEPS * N_PARTS,)),
        ],
        compiler_params=pltpu.CompilerParams(collective_id=0),
    )(gathered, maskf)


# device time: 26848 ns/iter; 1.0678x vs baseline; 1.0678x over previous
import jax
import jax.numpy as jnp
from jax import lax
from jax.experimental import pallas as pl
from jax.experimental.pallas import tpu as pltpu

N_DEV = 16
N_PARTS = 8
N_STEPS = 6

ORDERS = [
    [0, 1, 3, 2],
    [1, 2, 3, 0],
    [2, 0, 3, 1],
    [0, 2, 3, 1],
    [1, 0, 2, 3],
    [2, 1, 0, 3],
    [0, 1, 2, 3],
    [1, 2, 0, 3],
]


def kernel(ids, E):
    v_per, d = E.shape
    t = ids.shape[0]
    rows_per_part = t // N_PARTS
    h0 = rows_per_part // 2
    h1 = rows_per_part // 4

    my = lax.axis_index("i")
    local = ids - my * v_per
    mask = (local >= 0) & (local < v_per)
    safe = jnp.where(mask, local, 0)
    partial = jnp.where(mask[:, None], E[safe], 0.0).astype(jnp.float32)

    def body(p_ref, out_ref, s0, s1, s2, s3, send_sems, recv_sems):
        idx = lax.axis_index("i")
        z = idx // 4
        p = idx % 4
        x = jnp.where((p == 1) | (p == 2), 1, 0)
        y = jnp.where(p >= 2, 1, 0)

        def to_idx(xx, yy, zz):
            pp = jnp.where(yy == 1, 3 - xx, xx)
            return zz * 4 + pp

        partners = [
            to_idx(1 - x, y, z),
            to_idx(x, 1 - y, z),
            to_idx(x, y, jnp.bitwise_xor(z, 1)),
            to_idx(x, y, jnp.bitwise_xor(z, 2)),
        ]
        bits = [x, y, jnp.bitwise_and(z, 1), z // 2]
        scratches = [s0, s1, s2, s3]

        barrier_sem = pltpu.get_barrier_semaphore()
        for nbr in partners:
            pl.semaphore_signal(
                barrier_sem, inc=1,
                device_id=(nbr,), device_id_type=pl.DeviceIdType.MESH,
            )
        pl.semaphore_wait(barrier_sem, 4)

        def sem_idx(s, a):
            return s * N_PARTS + a

        off = [jnp.int32(a * rows_per_part) for a in range(N_PARTS)]

        def start(step, a):
            if step == 0 or step == 1:
                half = h0 if step == 0 else h1
                ax = ORDERS[a][step]
                b = bits[ax]
                my_off = off[a] + b * half
                pr_off = off[a] + (1 - b) * half
                src = out_ref.at[pl.ds(pr_off, half), :]
                dst = scratches[step].at[a]
                off[a] = my_off
                add_off = my_off
            elif step == 2 or step == 3:
                ax = ORDERS[a][step]
                src = out_ref.at[pl.ds(off[a], h1), :]
                dst = scratches[step].at[a]
                add_off = off[a]
            else:
                slot = 5 - step
                half = h1 if slot == 1 else h0
                ax = ORDERS[a][slot]
                src = out_ref.at[pl.ds(off[a], half), :]
                dst = out_ref.at[pl.ds(off[a], half), :]
                off[a] = off[a] - bits[ax] * half
                add_off = None
            rdma = pltpu.make_async_remote_copy(
                src_ref=src,
                dst_ref=dst,
                send_sem=send_sems.at[sem_idx(step, a)],
                recv_sem=recv_sems.at[sem_idx(step, a)],
                device_id=(partners[ax],),
                device_id_type=pl.DeviceIdType.MESH,
            )
            rdma.start()
            return rdma, add_off

        n_rows = [h0, h1, h1, h1, h1, h0]
        inflight = [start(0, a) for a in range(N_PARTS)]
        for step in range(1, N_STEPS):
            for a in range(N_PARTS):
                rdma, add_off = inflight[a]
                rdma.wait()
                if add_off is not None:
                    out_ref[pl.ds(add_off, n_rows[step - 1]), :] += (
                        scratches[step - 1][a, :, :]
                    )
                inflight[a] = start(step, a)
        for a in range(N_PARTS):
            inflight[a][0].wait()

    return pl.pallas_call(
        body,
        out_shape=jax.ShapeDtypeStruct((t, d), jnp.float32),
        in_specs=[pl.BlockSpec(memory_space=pltpu.VMEM)],
        out_specs=pl.BlockSpec(memory_space=pltpu.VMEM),
        input_output_aliases={0: 0},
        scratch_shapes=[
            pltpu.VMEM((N_PARTS, h0, d), jnp.float32),
            pltpu.VMEM((N_PARTS, h1, d), jnp.float32),
            pltpu.VMEM((N_PARTS, h1, d), jnp.float32),
            pltpu.VMEM((N_PARTS, h1, d), jnp.float32),
            pltpu.SemaphoreType.DMA((N_STEPS * N_PARTS,)),
            pltpu.SemaphoreType.DMA((N_STEPS * N_PARTS,)),
        ],
        compiler_params=pltpu.CompilerParams(collective_id=0),
    )(partial)
